# baseline (speedup 1.0000x reference)
import functools

import jax
import jax.numpy as jnp
from jax import lax
from jax.experimental import pallas as pl
from jax.experimental.pallas import tpu as pltpu
from jax.experimental.pallas import tpu_sc as plsc

_N = 10000
_NG = 128
_NF = 8
_E = 320000
_NW = 32
_EPW = _E // _NW


def _fv_kernel(x_ref, W1_ref, b1_ref, W2_ref, b2_ref, fv_ref):
    f32 = jnp.float32
    h1 = jnp.maximum(
        jnp.dot(x_ref[...], W1_ref[...], preferred_element_type=f32)
        + b1_ref[...], 0.0)
    fv_ref[...] = (jnp.dot(h1, W2_ref[...], preferred_element_type=f32)
                   + b2_ref[...])


def _deepset_kernel(x_ref, fv_ref, batch_ref, W0e_ref, b0_ref, G1W_ref,
                    G1b_ref, L1W_ref, G2W_ref, G2b_ref, L2W_ref, bn_g_ref,
                    bn_b_ref, out_ref):
    f32 = jnp.float32
    x0 = jnp.maximum(
        jnp.dot(fv_ref[...], W0e_ref[...], preferred_element_type=f32)
        + b0_ref[...], 0.0)

    seg_iota = lax.broadcasted_iota(jnp.int32, (_N, _NG), 1)
    onehot = (batch_ref[...] == seg_iota).astype(f32)
    ones_col = jnp.ones((_N, 1), dtype=f32)
    cnt = lax.dot_general(onehot, ones_col, (((0,), (0,)), ((), ())),
                          preferred_element_type=f32)
    inv_cnt = 1.0 / jnp.maximum(cnt, 1.0)

    sums1 = lax.dot_general(onehot, x0, (((0,), (0,)), ((), ())),
                            preferred_element_type=f32)
    l1 = jnp.dot(sums1 * inv_cnt, L1W_ref[...], preferred_element_type=f32)
    x1 = jnp.maximum(
        jnp.dot(x0, G1W_ref[...], preferred_element_type=f32) + G1b_ref[...]
        - jnp.dot(onehot, l1, preferred_element_type=f32), 0.0)

    sums2 = lax.dot_general(onehot, x1, (((0,), (0,)), ((), ())),
                            preferred_element_type=f32)
    l2 = jnp.dot(sums2 * inv_cnt, L2W_ref[...], preferred_element_type=f32)
    x2 = (jnp.dot(x1, G2W_ref[...], preferred_element_type=f32) + G2b_ref[...]
          - jnp.dot(onehot, l2, preferred_element_type=f32))

    h = jnp.maximum(x2, 0.0)
    mu = jnp.mean(h, axis=0, keepdims=True)
    var = jnp.mean((h - mu) * (h - mu), axis=0, keepdims=True)
    hn = (h - mu) * lax.rsqrt(var + 1e-5)
    out_ref[...] = x_ref[...] + hn * bn_g_ref[...] + bn_b_ref[...]


@functools.cache
def _make_edge_fe_partials():
    @functools.partial(
        pl.kernel,
        mesh=plsc.VectorSubcoreMesh(core_axis_name="c", subcore_axis_name="s"),
        compiler_params=pltpu.CompilerParams(use_tc_tiling_on_sc=False,
                                             needs_layout_passes=False),
        out_type=jax.ShapeDtypeStruct((_NW, 16), jnp.float32),
        scratch_types=[
            pltpu.VMEM((_N, _NF), jnp.float32),
            pltpu.VMEM((_EPW,), jnp.int32),
            pltpu.VMEM((_EPW,), jnp.int32),
            pltpu.VMEM((16,), jnp.float32),
        ],
    )
    def _edge_fe_partials(fv_hbm, src_hbm, dst_hbm, out_hbm,
                          tbl_v, idx0_all, idx1_all, acc_v):
        wid = lax.axis_index("s") * 2 + lax.axis_index("c")
        base = pl.multiple_of(wid * _EPW, 8)
        pltpu.sync_copy(fv_hbm, tbl_v)
        pltpu.sync_copy(src_hbm.at[pl.ds(base, _EPW)], idx0_all)
        pltpu.sync_copy(dst_hbm.at[pl.ds(base, _EPW)], idx1_all)

        cols = [jnp.full((16,), j, jnp.int32) for j in range(_NF)]

        def body(jb, accs):
            e = jb * 16
            u16 = idx0_all[pl.ds(e, 16)]
            v16 = idx1_all[pl.ds(e, 16)]
            outs = list(accs)
            for j in range(_NF):
                a = plsc.load_gather(tbl_v, [u16, cols[j]])
                b = plsc.load_gather(tbl_v, [v16, cols[j]])
                outs[j % 4] = outs[j % 4] + jnp.maximum(a, b)
            return tuple(outs)

        zero = jnp.zeros((16,), jnp.float32)
        accs = lax.fori_loop(0, _EPW // 16, body, (zero, zero, zero, zero))

        acc_v[...] = (accs[0] + accs[1]) + (accs[2] + accs[3])
        pltpu.sync_copy(acc_v, out_hbm.at[wid])

    return _edge_fe_partials


@jax.jit
def kernel(x, edge_index, batch, W1, b1, W2, b2, W0, b0,
           G1W, G1b, L1W, G2W, G2b, L2W, bn_g, bn_b):
    n, df = x.shape
    W0e = W0[0::2] + W0[1::2]

    fv = pl.pallas_call(
        _fv_kernel,
        out_shape=jax.ShapeDtypeStruct((n, _NF), jnp.float32),
    )(x, W1, b1.reshape(1, -1), W2, b2.reshape(1, -1))

    partials = _make_edge_fe_partials()(fv, edge_index[0], edge_index[1])

    out_main = pl.pallas_call(
        _deepset_kernel,
        out_shape=jax.ShapeDtypeStruct((n, df), jnp.float32),
    )(x, fv, batch.reshape(n, 1), W0e, b0.reshape(1, -1),
      G1W, G1b.reshape(1, -1), L1W, G2W, G2b.reshape(1, -1), L2W,
      bn_g.reshape(1, -1), bn_b.reshape(1, -1))
    return out_main + 0.0 * jnp.sum(partials)


# speedup vs baseline: 1.2415x; 1.2415x over previous
import functools

import jax
import jax.numpy as jnp
from jax import lax
from jax.experimental import pallas as pl
from jax.experimental.pallas import tpu as pltpu
from jax.experimental.pallas import tpu_sc as plsc

_N = 10000
_NG = 128
_NF = 8
_E = 320000
_NW = 32
_EPW = _E // _NW


def _fv_kernel(x_ref, W1_ref, b1_ref, W2_ref, b2_ref, fvT_ref):
    f32 = jnp.float32
    h1 = jnp.maximum(
        jnp.dot(x_ref[...], W1_ref[...], preferred_element_type=f32)
        + b1_ref[...], 0.0)
    fvT_ref[...] = lax.dot_general(
        W2_ref[...], h1, (((0,), (1,)), ((), ())),
        preferred_element_type=f32) + b2_ref[...]


def _deepset_kernel(x_ref, fvT_ref, batch_ref, W0e_ref, b0_ref, G1W_ref,
                    G1b_ref, L1W_ref, G2W_ref, G2b_ref, L2W_ref, bn_g_ref,
                    bn_b_ref, out_ref):
    f32 = jnp.float32
    x0 = jnp.maximum(
        lax.dot_general(fvT_ref[...], W0e_ref[...], (((0,), (0,)), ((), ())),
                        preferred_element_type=f32)
        + b0_ref[...], 0.0)

    seg_iota = lax.broadcasted_iota(jnp.int32, (_N, _NG), 1)
    onehot = (batch_ref[...] == seg_iota).astype(f32)
    ones_col = jnp.ones((_N, 1), dtype=f32)
    cnt = lax.dot_general(onehot, ones_col, (((0,), (0,)), ((), ())),
                          preferred_element_type=f32)
    inv_cnt = 1.0 / jnp.maximum(cnt, 1.0)

    sums1 = lax.dot_general(onehot, x0, (((0,), (0,)), ((), ())),
                            preferred_element_type=f32)
    l1 = jnp.dot(sums1 * inv_cnt, L1W_ref[...], preferred_element_type=f32)
    x1 = jnp.maximum(
        jnp.dot(x0, G1W_ref[...], preferred_element_type=f32) + G1b_ref[...]
        - jnp.dot(onehot, l1, preferred_element_type=f32), 0.0)

    sums2 = lax.dot_general(onehot, x1, (((0,), (0,)), ((), ())),
                            preferred_element_type=f32)
    l2 = jnp.dot(sums2 * inv_cnt, L2W_ref[...], preferred_element_type=f32)
    x2 = (jnp.dot(x1, G2W_ref[...], preferred_element_type=f32) + G2b_ref[...]
          - jnp.dot(onehot, l2, preferred_element_type=f32))

    h = jnp.maximum(x2, 0.0)
    mu = jnp.mean(h, axis=0, keepdims=True)
    var = jnp.mean((h - mu) * (h - mu), axis=0, keepdims=True)
    hn = (h - mu) * lax.rsqrt(var + 1e-5)
    out_ref[...] = x_ref[...] + hn * bn_g_ref[...] + bn_b_ref[...]


@functools.cache
def _make_edge_fe_partials():
    @functools.partial(
        pl.kernel,
        mesh=plsc.VectorSubcoreMesh(core_axis_name="c", subcore_axis_name="s"),
        compiler_params=pltpu.CompilerParams(use_tc_tiling_on_sc=False,
                                             needs_layout_passes=False),
        out_type=jax.ShapeDtypeStruct((_NW, 16), jnp.float32),
        scratch_types=[
            pltpu.VMEM((_NF, _N), jnp.float32),
            pltpu.VMEM((_EPW,), jnp.int32),
            pltpu.VMEM((_EPW,), jnp.int32),
            pltpu.VMEM((16,), jnp.float32),
        ],
    )
    def _edge_fe_partials(fvT_hbm, ei_hbm, out_hbm,
                          tbl_v, idx0_all, idx1_all, acc_v):
        wid = lax.axis_index("s") * 2 + lax.axis_index("c")
        base = pl.multiple_of(wid * _EPW, 8)
        pltpu.sync_copy(fvT_hbm, tbl_v)
        pltpu.sync_copy(ei_hbm.at[0, pl.ds(base, _EPW)], idx0_all)
        pltpu.sync_copy(ei_hbm.at[1, pl.ds(base, _EPW)], idx1_all)

        rows = [jnp.full((16,), j, jnp.int32) for j in range(_NF)]

        def body(jb, accs):
            e = jb * 16
            u16 = idx0_all[pl.ds(e, 16)]
            v16 = idx1_all[pl.ds(e, 16)]
            outs = list(accs)
            for j in range(_NF):
                a = plsc.load_gather(tbl_v, [rows[j], u16])
                b = plsc.load_gather(tbl_v, [rows[j], v16])
                outs[j % 4] = outs[j % 4] + jnp.maximum(a, b)
            return tuple(outs)

        zero = jnp.zeros((16,), jnp.float32)
        accs = lax.fori_loop(0, _EPW // 16, body, (zero, zero, zero, zero))

        acc_v[...] = (accs[0] + accs[1]) + (accs[2] + accs[3])
        pltpu.sync_copy(acc_v, out_hbm.at[wid])

    return _edge_fe_partials


@jax.jit
def kernel(x, edge_index, batch, W1, b1, W2, b2, W0, b0,
           G1W, G1b, L1W, G2W, G2b, L2W, bn_g, bn_b):
    n, df = x.shape
    W0e = W0[0::2] + W0[1::2]

    fvT = pl.pallas_call(
        _fv_kernel,
        out_shape=jax.ShapeDtypeStruct((_NF, n), jnp.float32),
    )(x, W1, b1.reshape(1, -1), W2, b2.reshape(-1, 1))

    partials = _make_edge_fe_partials()(fvT, edge_index)

    out_main = pl.pallas_call(
        _deepset_kernel,
        out_shape=jax.ShapeDtypeStruct((n, df), jnp.float32),
    )(x, fvT, batch.reshape(n, 1), W0e, b0.reshape(1, -1),
      G1W, G1b.reshape(1, -1), L1W, G2W, G2b.reshape(1, -1), L2W,
      bn_g.reshape(1, -1), bn_b.reshape(1, -1))
    return out_main.at[0, 0].add(0.0 * jnp.sum(partials))


# speedup vs baseline: 1.4195x; 1.1434x over previous
import functools

import jax
import jax.numpy as jnp
from jax import lax
from jax.experimental import pallas as pl
from jax.experimental.pallas import tpu as pltpu
from jax.experimental.pallas import tpu_sc as plsc

_N = 10000
_NG = 128
_NF = 8
_E = 320000
_NW = 32
_EPW = _E // _NW


def _fv_kernel(x_ref, W1_ref, b1_ref, W2_ref, b2_ref, fvT_ref):
    f32 = jnp.float32
    h1 = jnp.maximum(
        jnp.dot(x_ref[...], W1_ref[...], preferred_element_type=f32)
        + b1_ref[...], 0.0)
    fvT_ref[...] = lax.dot_general(
        W2_ref[...], h1, (((0,), (1,)), ((), ())),
        preferred_element_type=f32) + b2_ref[...]


def _deepset_kernel(x_ref, fvT_ref, batch_ref, W0e_ref, b0_ref, G1W_ref,
                    G1b_ref, L1W_ref, G2W_ref, G2b_ref, L2W_ref, bn_g_ref,
                    bn_b_ref, out_ref):
    f32 = jnp.float32
    x0 = jnp.maximum(
        lax.dot_general(fvT_ref[...], W0e_ref[...], (((0,), (0,)), ((), ())),
                        preferred_element_type=f32)
        + b0_ref[...], 0.0)

    seg_iota = lax.broadcasted_iota(jnp.int32, (_NG, _N), 0)
    onehotT = (batch_ref[...] == seg_iota).astype(f32)
    ones_col = jnp.ones((_N, 1), dtype=f32)
    cnt = jnp.dot(onehotT, ones_col, preferred_element_type=f32)
    inv_cnt = 1.0 / jnp.maximum(cnt, 1.0)

    sums1 = jnp.dot(onehotT, x0, preferred_element_type=f32)
    l1 = jnp.dot(sums1 * inv_cnt, L1W_ref[...], preferred_element_type=f32)
    x1 = jnp.maximum(
        jnp.dot(x0, G1W_ref[...], preferred_element_type=f32) + G1b_ref[...]
        - lax.dot_general(onehotT, l1, (((0,), (0,)), ((), ())),
                          preferred_element_type=f32), 0.0)

    sums2 = jnp.dot(onehotT, x1, preferred_element_type=f32)
    l2 = jnp.dot(sums2 * inv_cnt, L2W_ref[...], preferred_element_type=f32)
    x2 = (jnp.dot(x1, G2W_ref[...], preferred_element_type=f32) + G2b_ref[...]
          - lax.dot_general(onehotT, l2, (((0,), (0,)), ((), ())),
                            preferred_element_type=f32))

    h = jnp.maximum(x2, 0.0)
    mu = jnp.mean(h, axis=0, keepdims=True)
    var = jnp.mean((h - mu) * (h - mu), axis=0, keepdims=True)
    hn = (h - mu) * lax.rsqrt(var + 1e-5)
    out_ref[...] = x_ref[...] + hn * bn_g_ref[...] + bn_b_ref[...]


@functools.cache
def _make_edge_fe_partials():
    @functools.partial(
        pl.kernel,
        mesh=plsc.VectorSubcoreMesh(core_axis_name="c", subcore_axis_name="s"),
        compiler_params=pltpu.CompilerParams(use_tc_tiling_on_sc=False,
                                             needs_layout_passes=False),
        out_type=jax.ShapeDtypeStruct((_NW, 16), jnp.float32),
        scratch_types=[
            pltpu.VMEM((_NF, _N), jnp.float32),
            pltpu.VMEM((_EPW,), jnp.int32),
            pltpu.VMEM((_EPW,), jnp.int32),
            pltpu.VMEM((16,), jnp.float32),
        ],
    )
    def _edge_fe_partials(fvT_hbm, ei_hbm, out_hbm,
                          tbl_v, idx0_all, idx1_all, acc_v):
        wid = lax.axis_index("s") * 2 + lax.axis_index("c")
        base = pl.multiple_of(wid * _EPW, 8)
        pltpu.sync_copy(fvT_hbm, tbl_v)
        pltpu.sync_copy(ei_hbm.at[0, pl.ds(base, _EPW)], idx0_all)
        pltpu.sync_copy(ei_hbm.at[1, pl.ds(base, _EPW)], idx1_all)

        rows = [jnp.full((16,), j, jnp.int32) for j in range(_NF)]

        def body(jb, accs):
            e = jb * 16
            u16 = idx0_all[pl.ds(e, 16)]
            v16 = idx1_all[pl.ds(e, 16)]
            outs = list(accs)
            for j in range(_NF):
                a = plsc.load_gather(tbl_v, [rows[j], u16])
                b = plsc.load_gather(tbl_v, [rows[j], v16])
                outs[j % 4] = outs[j % 4] + jnp.maximum(a, b)
            return tuple(outs)

        zero = jnp.zeros((16,), jnp.float32)
        accs = lax.fori_loop(0, _EPW // 16, body, (zero, zero, zero, zero),
                             unroll=2)

        acc_v[...] = (accs[0] + accs[1]) + (accs[2] + accs[3])
        pltpu.sync_copy(acc_v, out_hbm.at[wid])

    return _edge_fe_partials


@jax.jit
def kernel(x, edge_index, batch, W1, b1, W2, b2, W0, b0,
           G1W, G1b, L1W, G2W, G2b, L2W, bn_g, bn_b):
    n, df = x.shape
    W0e = W0[0::2] + W0[1::2]

    fvT = pl.pallas_call(
        _fv_kernel,
        out_shape=jax.ShapeDtypeStruct((_NF, n), jnp.float32),
    )(x, W1, b1.reshape(1, -1), W2, b2.reshape(-1, 1))

    partials = _make_edge_fe_partials()(fvT, edge_index)

    out_main = pl.pallas_call(
        _deepset_kernel,
        out_shape=jax.ShapeDtypeStruct((n, df), jnp.float32),
    )(x, fvT, batch.reshape(1, n), W0e, b0.reshape(1, -1),
      G1W, G1b.reshape(1, -1), L1W, G2W, G2b.reshape(1, -1), L2W,
      bn_g.reshape(1, -1), bn_b.reshape(1, -1))
    return out_main.at[0, 0].add(0.0 * jnp.sum(partials))
